# Initial kernel scaffold; baseline (speedup 1.0000x reference)
import jax
import jax.numpy as jnp
from jax.experimental import pallas as pl


def kernel(features_1, edge_index_1, features_2, edge_index_2, hb, W_pre, b_pre, conv_W1, conv_b1, conv_W2, conv_b2, Wp1, bp1, Wp2, bp2):
    raise NotImplementedError("write your pallas kernel here")


# speedup vs baseline: 1.1116x; 1.1116x over previous
import functools

import jax
import jax.numpy as jnp
from jax import lax
from jax.experimental import pallas as pl
from jax.experimental.pallas import tpu as pltpu
from jax.experimental.pallas import tpu_sc as plsc

N = 10000
E = 160000
IN_DIM = 256
H = 512
N_LAYERS = 8

NWORK = 32
PASSES = 2
NBUCKET = NWORK * PASSES
BR = 160
NPAD = NBUCKET * BR
ACC_ROWS = 168
TRASH = 160
B = 64
PADROWS = E // B + NBUCKET

_f32 = jnp.float32
_i32 = jnp.int32


def _prep_edges(edge_index):
    src = edge_index[0].astype(_i32)
    dst = edge_index[1].astype(_i32)
    q = dst // BR
    order = jnp.argsort(q, stable=True)
    qs = q[order]
    counts = jnp.zeros((NBUCKET,), _i32).at[q].add(1)
    nb = (counts + B - 1) // B
    pstarts = jnp.concatenate([jnp.zeros((1,), _i32),
                               jnp.cumsum(nb * B)[:-1].astype(_i32)])
    ustarts = jnp.concatenate([jnp.zeros((1,), _i32),
                               jnp.cumsum(counts)[:-1].astype(_i32)])
    rank = jnp.arange(E, dtype=_i32) - ustarts[qs]
    pos = pstarts[qs] + rank
    psrc = jnp.zeros((PADROWS * B,), _i32).at[pos].set(src[order])
    pdst = jnp.full((PADROWS * B,), TRASH, _i32).at[pos].set(
        dst[order] - qs * BR)
    pad16 = jnp.zeros((16,), _i32)
    row0 = jnp.concatenate([pstarts // B, pad16])
    nb_p = jnp.concatenate([nb, pad16])
    return (psrc, pdst, row0, nb_p)


def _sc_scatter_body(x_hbm, srcp_hbm, dstp_hbm, row0_hbm, nb_hbm, out_hbm,
                     src_v, dst_v, rows_v, row0_v, nb_v, acc_v, gsem):
    core = lax.axis_index("c")
    sid = lax.axis_index("s")
    wid = core * 16 + sid

    pltpu.sync_copy(row0_hbm, row0_v)
    pltpu.sync_copy(nb_hbm, nb_v)
    zero16 = jnp.zeros((16,), _f32)

    for k in range(PASSES):
        bucket = k * NWORK + wid

        def zrow(r, carry):
            for j in range(H // 16):
                acc_v[r, pl.ds(j * 16, 16)] = zero16
            return carry

        lax.fori_loop(0, BR, zrow, 0)

        r0 = row0_v[pl.ds(bucket, 16)][0]
        my_nb = nb_v[pl.ds(bucket, 16)][0]

        def body(b, carry):
            r = r0 + b
            pltpu.sync_copy(srcp_hbm.at[pl.ds(r * B, B)],
                            src_v.at[pl.ds(0, B)])
            pltpu.async_copy(x_hbm.at[src_v.at[pl.ds(0, B)]], rows_v,
                             gsem).wait()
            pltpu.sync_copy(dstp_hbm.at[pl.ds(r * B, B)],
                            dst_v.at[pl.ds(0, B)])

            def edge(e, c2):
                dloc = dst_v[pl.ds(e, 16)][0]
                for j in range(H // 16):
                    plsc.addupdate(acc_v.at[dloc, pl.ds(j * 16, 16)],
                                   rows_v[e, pl.ds(j * 16, 16)])
                return c2

            lax.fori_loop(0, B, edge, 0)
            return carry

        lax.fori_loop(0, my_nb, body, 0)

        pltpu.sync_copy(acc_v.at[pl.ds(0, BR)],
                        out_hbm.at[pl.ds(bucket * BR, BR)])


@functools.cache
def _get_sc_scatter():
    return pl.kernel(
        _sc_scatter_body,
        out_type=jax.ShapeDtypeStruct((NPAD, H), _f32),
        mesh=plsc.VectorSubcoreMesh(core_axis_name="c", subcore_axis_name="s"),
        scratch_types=[
            pltpu.VMEM((B + 16,), _i32),
            pltpu.VMEM((B + 16,), _i32),
            pltpu.VMEM((B, H), _f32),
            pltpu.VMEM((NBUCKET + 16,), _i32),
            pltpu.VMEM((NBUCKET + 16,), _i32),
            pltpu.VMEM((ACC_ROWS, H), _f32),
            pltpu.SemaphoreType.DMA,
        ],
    )


def _sc_scatter(x, psrc, pdst, row0, nb):
    return _get_sc_scatter()(x, psrc, pdst, row0, nb)


R = 2000
GRID = N // R

_dot = functools.partial(jnp.dot, preferred_element_type=_f32,
                         precision=lax.Precision.HIGHEST)


def _pre_body(f_ref, w_ref, b_ref, x_ref, p_ref):
    x = _dot(f_ref[...], w_ref[...]) + b_ref[...]
    x_ref[...] = x

    @pl.when(pl.program_id(0) == 0)
    def _():
        p_ref[...] = jnp.zeros_like(p_ref)

    p_ref[...] += jnp.sum(x, axis=0, keepdims=True)


def _tc_pre(f, w, b):
    return pl.pallas_call(
        _pre_body,
        grid=(GRID,),
        in_specs=[
            pl.BlockSpec((R, IN_DIM), lambda i: (i, 0)),
            pl.BlockSpec((IN_DIM, H), lambda i: (0, 0)),
            pl.BlockSpec((1, H), lambda i: (0, 0)),
        ],
        out_specs=[
            pl.BlockSpec((R, H), lambda i: (i, 0)),
            pl.BlockSpec((1, H), lambda i: (0, 0)),
        ],
        out_shape=[
            jax.ShapeDtypeStruct((N, H), _f32),
            jax.ShapeDtypeStruct((1, H), _f32),
        ],
    )(f, w, b)


def _layer_even_body(x_ref, a_ref, w1_ref, b1_ref, w2_ref, b2_ref,
                     xo_ref, p_ref):
    h = x_ref[...] + a_ref[...]
    t = jnp.maximum(_dot(h, w1_ref[...]) + b1_ref[...], 0.0)
    o = _dot(t, w2_ref[...]) + b2_ref[...]
    xo = jnp.maximum(o, 0.0)
    xo_ref[...] = xo

    @pl.when(pl.program_id(0) == 0)
    def _():
        p_ref[...] = jnp.zeros_like(p_ref)

    p_ref[...] += jnp.sum(xo, axis=0, keepdims=True)


def _layer_odd_body(x_ref, a_ref, r_ref, w1_ref, b1_ref, w2_ref, b2_ref,
                    xo_ref, ro_ref, p_ref):
    h = x_ref[...] + a_ref[...]
    t = jnp.maximum(_dot(h, w1_ref[...]) + b1_ref[...], 0.0)
    o = _dot(t, w2_ref[...]) + b2_ref[...] + r_ref[...]
    ro_ref[...] = o
    xo = jnp.maximum(o, 0.0)
    xo_ref[...] = xo

    @pl.when(pl.program_id(0) == 0)
    def _():
        p_ref[...] = jnp.zeros_like(p_ref)

    p_ref[...] += jnp.sum(xo, axis=0, keepdims=True)


_NODE_SPEC = pl.BlockSpec((R, H), lambda i: (i, 0))
_W_SPEC = pl.BlockSpec((H, H), lambda i: (0, 0))
_B_SPEC = pl.BlockSpec((1, H), lambda i: (0, 0))


def _tc_layer_even(x, agg, w1, b1, w2, b2):
    return pl.pallas_call(
        _layer_even_body,
        grid=(GRID,),
        in_specs=[_NODE_SPEC, _NODE_SPEC, _W_SPEC, _B_SPEC, _W_SPEC, _B_SPEC],
        out_specs=[_NODE_SPEC, _B_SPEC],
        out_shape=[
            jax.ShapeDtypeStruct((N, H), _f32),
            jax.ShapeDtypeStruct((1, H), _f32),
        ],
    )(x, agg, w1, b1, w2, b2)


def _tc_layer_odd(x, agg, res, w1, b1, w2, b2):
    return pl.pallas_call(
        _layer_odd_body,
        grid=(GRID,),
        in_specs=[_NODE_SPEC, _NODE_SPEC, _NODE_SPEC, _W_SPEC, _B_SPEC,
                  _W_SPEC, _B_SPEC],
        out_specs=[_NODE_SPEC, _NODE_SPEC, _B_SPEC],
        out_shape=[
            jax.ShapeDtypeStruct((N, H), _f32),
            jax.ShapeDtypeStruct((N, H), _f32),
            jax.ShapeDtypeStruct((1, H), _f32),
        ],
    )(x, agg, res, w1, b1, w2, b2)


def _head_body(p1_ref, p2_ref, wp1_ref, bp1_ref, wp2_ref, bp2_ref, hb_ref,
               s_ref, g_ref):
    def mlp(p):
        t = jnp.maximum(_dot(p, wp1_ref[...]) + bp1_ref[...], 0.0)
        return _dot(t, wp2_ref[...]) + bp2_ref[...]

    d = mlp(p1_ref[...]) - mlp(p2_ref[...])
    s = jnp.sqrt(jnp.sum(d * d))
    s_ref[0, 0] = s
    g_ref[0, 0] = s * hb_ref[0, 0]


def _tc_head(p1, p2, wp1, bp1, wp2, bp2, hb_arr):
    cat = H * (N_LAYERS + 1)
    return pl.pallas_call(
        _head_body,
        in_specs=[
            pl.BlockSpec((1, cat), lambda: (0, 0)),
            pl.BlockSpec((1, cat), lambda: (0, 0)),
            pl.BlockSpec((cat, H), lambda: (0, 0)),
            pl.BlockSpec((1, H), lambda: (0, 0)),
            pl.BlockSpec((H, H), lambda: (0, 0)),
            pl.BlockSpec((1, H), lambda: (0, 0)),
            pl.BlockSpec(memory_space=pltpu.SMEM),
        ],
        out_specs=[
            pl.BlockSpec(memory_space=pltpu.SMEM),
            pl.BlockSpec(memory_space=pltpu.SMEM),
        ],
        out_shape=[
            jax.ShapeDtypeStruct((1, 1), _f32),
            jax.ShapeDtypeStruct((1, 1), _f32),
        ],
    )(p1, p2, wp1, bp1, wp2, bp2, hb_arr)


def kernel(features_1, edge_index_1, features_2, edge_index_2, hb, W_pre,
           b_pre, conv_W1, conv_b1, conv_W2, conv_b2, Wp1, bp1, Wp2, bp2):
    b_pre2 = b_pre.reshape(1, H)
    bp1_2 = bp1.reshape(1, H)
    bp2_2 = bp2.reshape(1, H)

    prep1 = _prep_edges(edge_index_1)
    prep2 = _prep_edges(edge_index_2)

    x1, p1_0 = _tc_pre(features_1, W_pre, b_pre2)
    x2, p2_0 = _tc_pre(features_2, W_pre, b_pre2)
    res1, res2 = x1, x2
    pooled1, pooled2 = [p1_0], [p2_0]

    for i in range(N_LAYERS):
        w1 = conv_W1[i]
        b1 = conv_b1[i].reshape(1, H)
        w2 = conv_W2[i]
        b2 = conv_b2[i].reshape(1, H)
        agg1 = _sc_scatter(x1, *prep1)
        agg2 = _sc_scatter(x2, *prep2)
        if i & 1:
            x1, res1, p1 = _tc_layer_odd(x1, agg1, res1, w1, b1, w2, b2)
            x2, res2, p2 = _tc_layer_odd(x2, agg2, res2, w1, b1, w2, b2)
        else:
            x1, p1 = _tc_layer_even(x1, agg1, w1, b1, w2, b2)
            x2, p2 = _tc_layer_even(x2, agg2, w1, b1, w2, b2)
        pooled1.append(p1)
        pooled2.append(p2)

    pc1 = jnp.concatenate(pooled1, axis=1)
    pc2 = jnp.concatenate(pooled2, axis=1)
    hb_arr = jnp.asarray(hb, _f32).reshape(1, 1)
    s11, g11 = _tc_head(pc1, pc2, Wp1, bp1_2, Wp2, bp2_2, hb_arr)
    return (s11.reshape(-1), g11.reshape(-1))


# speedup vs baseline: 1.1564x; 1.0403x over previous
import functools

import jax
import jax.numpy as jnp
from jax import lax
from jax.experimental import pallas as pl
from jax.experimental.pallas import tpu as pltpu
from jax.experimental.pallas import tpu_sc as plsc

N = 10000
E = 160000
IN_DIM = 256
H = 512
N_LAYERS = 8

NWORK = 32
PASSES = 2
NBUCKET = NWORK * PASSES
BR = 160
NPAD = NBUCKET * BR
ACC_ROWS = 168
TRASH = 160
B = 64
PADROWS = E // B + NBUCKET

_f32 = jnp.float32
_i32 = jnp.int32


def _prep_edges(edge_index):
    src = edge_index[0].astype(_i32)
    dst = edge_index[1].astype(_i32)
    q = dst // BR
    order = jnp.argsort(q, stable=True)
    qs = q[order]
    counts = jnp.zeros((NBUCKET,), _i32).at[q].add(1)
    nb = (counts + B - 1) // B
    pstarts = jnp.concatenate([jnp.zeros((1,), _i32),
                               jnp.cumsum(nb * B)[:-1].astype(_i32)])
    ustarts = jnp.concatenate([jnp.zeros((1,), _i32),
                               jnp.cumsum(counts)[:-1].astype(_i32)])
    rank = jnp.arange(E, dtype=_i32) - ustarts[qs]
    pos = pstarts[qs] + rank
    psrc = jnp.zeros((PADROWS * B,), _i32).at[pos].set(src[order])
    pdst = jnp.full((PADROWS * B,), TRASH, _i32).at[pos].set(
        dst[order] - qs * BR)
    pad16 = jnp.zeros((16,), _i32)
    row0 = jnp.concatenate([pstarts // B, pad16])
    nb_p = jnp.concatenate([nb, pad16])
    return (psrc, pdst, row0, nb_p)


def _sc_scatter_body(x_hbm, srcp_hbm, dstp_hbm, row0_hbm, nb_hbm, out_hbm,
                     src_v, dst_v, rows_v, row0_v, nb_v, acc_v, gsem):
    core = lax.axis_index("c")
    sid = lax.axis_index("s")
    wid = core * 16 + sid

    pltpu.sync_copy(row0_hbm, row0_v)
    pltpu.sync_copy(nb_hbm, nb_v)
    zero16 = jnp.zeros((16,), _f32)

    acc1 = acc_v

    for k in range(PASSES):
        bucket = k * NWORK + wid

        def zrow(r, carry):
            for j in range(H // 16):
                acc1[pl.ds(r * H + j * 16, 16)] = zero16
            return carry

        lax.fori_loop(0, BR, zrow, 0)

        r0 = row0_v[pl.ds(bucket, 16)][0]
        my_nb = nb_v[pl.ds(bucket, 16)][0]

        def body(b, carry):
            r = r0 + b
            pltpu.sync_copy(srcp_hbm.at[pl.ds(r * B, B)],
                            src_v.at[pl.ds(0, B)])
            pltpu.async_copy(x_hbm.at[src_v.at[pl.ds(0, B)]], rows_v,
                             gsem).wait()
            pltpu.sync_copy(dstp_hbm.at[pl.ds(r * B, B)],
                            dst_v.at[pl.ds(0, B)])

            def group(g, c2):
                dvec = dst_v[pl.ds(4 * g, 16)]
                for i in range(4):
                    abase = dvec[i] * H
                    for j in range(H // 16):
                        plsc.addupdate(
                            acc1.at[pl.ds(abase + j * 16, 16)],
                            rows_v[4 * g + i, pl.ds(j * 16, 16)])
                return c2

            lax.fori_loop(0, B // 4, group, 0)
            return carry

        lax.fori_loop(0, my_nb, body, 0)

        pltpu.sync_copy(acc1.at[pl.ds(0, BR * H)],
                        out_hbm.at[pl.ds(bucket * (BR * H), BR * H)])


@functools.cache
def _get_sc_scatter():
    return pl.kernel(
        _sc_scatter_body,
        out_type=jax.ShapeDtypeStruct((NPAD * H,), _f32),
        mesh=plsc.VectorSubcoreMesh(core_axis_name="c", subcore_axis_name="s"),
        scratch_types=[
            pltpu.VMEM((B + 16,), _i32),
            pltpu.VMEM((B + 16,), _i32),
            pltpu.VMEM((B, H), _f32),
            pltpu.VMEM((NBUCKET + 16,), _i32),
            pltpu.VMEM((NBUCKET + 16,), _i32),
            pltpu.VMEM((ACC_ROWS * H,), _f32),
            pltpu.SemaphoreType.DMA,
        ],
    )


def _sc_scatter(x, psrc, pdst, row0, nb):
    return _get_sc_scatter()(x, psrc, pdst, row0, nb).reshape(NPAD, H)


R = 2000
GRID = N // R

_dot = functools.partial(jnp.dot, preferred_element_type=_f32,
                         precision=lax.Precision.DEFAULT)


def _pre_body(f_ref, w_ref, b_ref, x_ref, p_ref):
    x = _dot(f_ref[...], w_ref[...]) + b_ref[...]
    x_ref[...] = x

    @pl.when(pl.program_id(0) == 0)
    def _():
        p_ref[...] = jnp.zeros_like(p_ref)

    p_ref[...] += jnp.sum(x, axis=0, keepdims=True)


def _tc_pre(f, w, b):
    return pl.pallas_call(
        _pre_body,
        grid=(GRID,),
        in_specs=[
            pl.BlockSpec((R, IN_DIM), lambda i: (i, 0)),
            pl.BlockSpec((IN_DIM, H), lambda i: (0, 0)),
            pl.BlockSpec((1, H), lambda i: (0, 0)),
        ],
        out_specs=[
            pl.BlockSpec((R, H), lambda i: (i, 0)),
            pl.BlockSpec((1, H), lambda i: (0, 0)),
        ],
        out_shape=[
            jax.ShapeDtypeStruct((N, H), _f32),
            jax.ShapeDtypeStruct((1, H), _f32),
        ],
    )(f, w, b)


def _layer_even_body(x_ref, a_ref, w1_ref, b1_ref, w2_ref, b2_ref,
                     xo_ref, p_ref):
    h = x_ref[...] + a_ref[...]
    t = jnp.maximum(_dot(h, w1_ref[...]) + b1_ref[...], 0.0)
    o = _dot(t, w2_ref[...]) + b2_ref[...]
    xo = jnp.maximum(o, 0.0)
    xo_ref[...] = xo

    @pl.when(pl.program_id(0) == 0)
    def _():
        p_ref[...] = jnp.zeros_like(p_ref)

    p_ref[...] += jnp.sum(xo, axis=0, keepdims=True)


def _layer_odd_body(x_ref, a_ref, r_ref, w1_ref, b1_ref, w2_ref, b2_ref,
                    xo_ref, ro_ref, p_ref):
    h = x_ref[...] + a_ref[...]
    t = jnp.maximum(_dot(h, w1_ref[...]) + b1_ref[...], 0.0)
    o = _dot(t, w2_ref[...]) + b2_ref[...] + r_ref[...]
    ro_ref[...] = o
    xo = jnp.maximum(o, 0.0)
    xo_ref[...] = xo

    @pl.when(pl.program_id(0) == 0)
    def _():
        p_ref[...] = jnp.zeros_like(p_ref)

    p_ref[...] += jnp.sum(xo, axis=0, keepdims=True)


_NODE_SPEC = pl.BlockSpec((R, H), lambda i: (i, 0))
_W_SPEC = pl.BlockSpec((H, H), lambda i: (0, 0))
_B_SPEC = pl.BlockSpec((1, H), lambda i: (0, 0))


def _tc_layer_even(x, agg, w1, b1, w2, b2):
    return pl.pallas_call(
        _layer_even_body,
        grid=(GRID,),
        in_specs=[_NODE_SPEC, _NODE_SPEC, _W_SPEC, _B_SPEC, _W_SPEC, _B_SPEC],
        out_specs=[_NODE_SPEC, _B_SPEC],
        out_shape=[
            jax.ShapeDtypeStruct((N, H), _f32),
            jax.ShapeDtypeStruct((1, H), _f32),
        ],
    )(x, agg, w1, b1, w2, b2)


def _tc_layer_odd(x, agg, res, w1, b1, w2, b2):
    return pl.pallas_call(
        _layer_odd_body,
        grid=(GRID,),
        in_specs=[_NODE_SPEC, _NODE_SPEC, _NODE_SPEC, _W_SPEC, _B_SPEC,
                  _W_SPEC, _B_SPEC],
        out_specs=[_NODE_SPEC, _NODE_SPEC, _B_SPEC],
        out_shape=[
            jax.ShapeDtypeStruct((N, H), _f32),
            jax.ShapeDtypeStruct((N, H), _f32),
            jax.ShapeDtypeStruct((1, H), _f32),
        ],
    )(x, agg, res, w1, b1, w2, b2)


def _head_body(p1_ref, p2_ref, wp1_ref, bp1_ref, wp2_ref, bp2_ref, hb_ref,
               s_ref, g_ref):
    def mlp(p):
        t = jnp.maximum(_dot(p, wp1_ref[...]) + bp1_ref[...], 0.0)
        return _dot(t, wp2_ref[...]) + bp2_ref[...]

    d = mlp(p1_ref[...]) - mlp(p2_ref[...])
    s = jnp.sqrt(jnp.sum(d * d))
    s_ref[0, 0] = s
    g_ref[0, 0] = s * hb_ref[0, 0]


def _tc_head(p1, p2, wp1, bp1, wp2, bp2, hb_arr):
    cat = H * (N_LAYERS + 1)
    return pl.pallas_call(
        _head_body,
        in_specs=[
            pl.BlockSpec((1, cat), lambda: (0, 0)),
            pl.BlockSpec((1, cat), lambda: (0, 0)),
            pl.BlockSpec((cat, H), lambda: (0, 0)),
            pl.BlockSpec((1, H), lambda: (0, 0)),
            pl.BlockSpec((H, H), lambda: (0, 0)),
            pl.BlockSpec((1, H), lambda: (0, 0)),
            pl.BlockSpec(memory_space=pltpu.SMEM),
        ],
        out_specs=[
            pl.BlockSpec(memory_space=pltpu.SMEM),
            pl.BlockSpec(memory_space=pltpu.SMEM),
        ],
        out_shape=[
            jax.ShapeDtypeStruct((1, 1), _f32),
            jax.ShapeDtypeStruct((1, 1), _f32),
        ],
    )(p1, p2, wp1, bp1, wp2, bp2, hb_arr)


def kernel(features_1, edge_index_1, features_2, edge_index_2, hb, W_pre,
           b_pre, conv_W1, conv_b1, conv_W2, conv_b2, Wp1, bp1, Wp2, bp2):
    b_pre2 = b_pre.reshape(1, H)
    bp1_2 = bp1.reshape(1, H)
    bp2_2 = bp2.reshape(1, H)

    prep1 = _prep_edges(edge_index_1)
    prep2 = _prep_edges(edge_index_2)

    x1, p1_0 = _tc_pre(features_1, W_pre, b_pre2)
    x2, p2_0 = _tc_pre(features_2, W_pre, b_pre2)
    res1, res2 = x1, x2
    pooled1, pooled2 = [p1_0], [p2_0]

    for i in range(N_LAYERS):
        w1 = conv_W1[i]
        b1 = conv_b1[i].reshape(1, H)
        w2 = conv_W2[i]
        b2 = conv_b2[i].reshape(1, H)
        agg1 = _sc_scatter(x1, *prep1)
        agg2 = _sc_scatter(x2, *prep2)
        if i & 1:
            x1, res1, p1 = _tc_layer_odd(x1, agg1, res1, w1, b1, w2, b2)
            x2, res2, p2 = _tc_layer_odd(x2, agg2, res2, w1, b1, w2, b2)
        else:
            x1, p1 = _tc_layer_even(x1, agg1, w1, b1, w2, b2)
            x2, p2 = _tc_layer_even(x2, agg2, w1, b1, w2, b2)
        pooled1.append(p1)
        pooled2.append(p2)

    pc1 = jnp.concatenate(pooled1, axis=1)
    pc2 = jnp.concatenate(pooled2, axis=1)
    hb_arr = jnp.asarray(hb, _f32).reshape(1, 1)
    s11, g11 = _tc_head(pc1, pc2, Wp1, bp1_2, Wp2, bp2_2, hb_arr)
    return (s11.reshape(-1), g11.reshape(-1))


# speedup vs baseline: 1.4212x; 1.2290x over previous
import functools

import jax
import jax.numpy as jnp
from jax import lax
from jax.experimental import pallas as pl
from jax.experimental.pallas import tpu as pltpu
from jax.experimental.pallas import tpu_sc as plsc

N = 10000
E = 160000
IN_DIM = 256
H = 512
N_LAYERS = 8

NWORK = 32
PASSES = 2
NBUCKET = NWORK * PASSES
BR = 160
NPAD = NBUCKET * BR
ACC_ROWS = 168
TRASH = 160
B = 32
CHUNKB = 128
PADTOT = E + 8448

_f32 = jnp.float32
_i32 = jnp.int32


def _prep_edges(edge_index):
    src = edge_index[0].astype(_i32)
    dst = edge_index[1].astype(_i32)
    q = dst // BR
    order = jnp.argsort(q, stable=True)
    qs = q[order]
    counts = jnp.zeros((NBUCKET,), _i32).at[q].add(1)
    padded = ((counts + 2 * B - 1) // (2 * B)) * (2 * B)
    nb = padded // B
    pstarts = jnp.concatenate([jnp.zeros((1,), _i32),
                               jnp.cumsum(padded)[:-1].astype(_i32)])
    ustarts = jnp.concatenate([jnp.zeros((1,), _i32),
                               jnp.cumsum(counts)[:-1].astype(_i32)])
    rank = jnp.arange(E, dtype=_i32) - ustarts[qs]
    pos = pstarts[qs] + rank
    psrc = jnp.zeros((PADTOT,), _i32).at[pos].set(src[order])
    pdst = jnp.full((PADTOT,), TRASH, _i32).at[pos].set(
        dst[order] - qs * BR)
    pad16 = jnp.zeros((16,), _i32)
    row0 = jnp.concatenate([pstarts // B, pad16])
    nb_p = jnp.concatenate([nb, pad16])
    return (psrc, pdst, row0, nb_p)


def _sc_scatter_body(x_hbm, srcp_hbm, dstp_hbm, row0_hbm, nb_hbm, out_hbm,
                     src_v, dst_v, rows0_v, rows1_v, row0_v, nb_v, acc1,
                     sem0, sem1):
    core = lax.axis_index("c")
    sid = lax.axis_index("s")
    wid = core * 16 + sid

    pltpu.sync_copy(row0_hbm, row0_v)
    pltpu.sync_copy(nb_hbm, nb_v)
    zero16 = jnp.zeros((16,), _f32)

    def gather(b_in_chunk, buf, sem):
        pltpu.async_copy(
            x_hbm.at[src_v.at[pl.ds(b_in_chunk * B, B)]], buf, sem)

    def gwait(buf, sem):
        pltpu.make_async_copy(x_hbm.at[src_v.at[pl.ds(0, B)]], buf,
                              sem).wait()

    def accum(buf, dboff):
        def group(g, c2):
            dvec = dst_v[pl.ds(dboff + 4 * g, 16)]
            for i in range(4):
                abase = dvec[i] * H
                for j in range(H // 16):
                    plsc.addupdate(
                        acc1.at[pl.ds(abase + j * 16, 16)],
                        buf[4 * g + i, pl.ds(j * 16, 16)])
            return c2

        lax.fori_loop(0, B // 4, group, 0)

    for k in range(PASSES):
        bucket = k * NWORK + wid

        def zrow(r, carry):
            for j in range(H // 16):
                acc1[pl.ds(r * H + j * 16, 16)] = zero16
            return carry

        lax.fori_loop(0, BR, zrow, 0)

        r0 = row0_v[pl.ds(bucket, 16)][0]
        my_nb = nb_v[pl.ds(bucket, 16)][0]

        def chunk(c, carry):
            cb0 = c * CHUNKB
            ebase = (r0 + cb0) * B
            pltpu.sync_copy(srcp_hbm.at[pl.ds(ebase, CHUNKB * B)], src_v)
            pltpu.sync_copy(dstp_hbm.at[pl.ds(ebase, CHUNKB * B)],
                            dst_v.at[pl.ds(0, CHUNKB * B)])
            cnb = jnp.minimum(CHUNKB, my_nb - cb0)

            gather(0, rows0_v, sem0)

            def pair(g, c2):
                gather(2 * g + 1, rows1_v, sem1)
                gwait(rows0_v, sem0)
                accum(rows0_v, 2 * g * B)

                @pl.when(2 * g + 2 < cnb)
                def _():
                    gather(2 * g + 2, rows0_v, sem0)

                gwait(rows1_v, sem1)
                accum(rows1_v, (2 * g + 1) * B)
                return c2

            lax.fori_loop(0, cnb // 2, pair, 0)
            return carry

        lax.fori_loop(0, (my_nb + CHUNKB - 1) // CHUNKB, chunk, 0)

        pltpu.sync_copy(acc1.at[pl.ds(0, BR * H)],
                        out_hbm.at[pl.ds(bucket * (BR * H), BR * H)])


@functools.cache
def _get_sc_scatter():
    return pl.kernel(
        _sc_scatter_body,
        out_type=jax.ShapeDtypeStruct((NPAD * H,), _f32),
        mesh=plsc.VectorSubcoreMesh(core_axis_name="c", subcore_axis_name="s"),
        scratch_types=[
            pltpu.VMEM((CHUNKB * B,), _i32),
            pltpu.VMEM((CHUNKB * B + 16,), _i32),
            pltpu.VMEM((B, H), _f32),
            pltpu.VMEM((B, H), _f32),
            pltpu.VMEM((NBUCKET + 16,), _i32),
            pltpu.VMEM((NBUCKET + 16,), _i32),
            pltpu.VMEM((ACC_ROWS * H,), _f32),
            pltpu.SemaphoreType.DMA,
            pltpu.SemaphoreType.DMA,
        ],
    )


def _sc_scatter(x, psrc, pdst, row0, nb):
    return _get_sc_scatter()(x, psrc, pdst, row0, nb).reshape(NPAD, H)


R = 2000
GRID = N // R

_dot = functools.partial(jnp.dot, preferred_element_type=_f32,
                         precision=lax.Precision.DEFAULT)


def _pre_body(f_ref, w_ref, b_ref, x_ref, p_ref):
    x = _dot(f_ref[...], w_ref[...]) + b_ref[...]
    x_ref[...] = x

    @pl.when(pl.program_id(0) == 0)
    def _():
        p_ref[...] = jnp.zeros_like(p_ref)

    p_ref[...] += jnp.sum(x, axis=0, keepdims=True)


def _tc_pre(f, w, b):
    return pl.pallas_call(
        _pre_body,
        grid=(GRID,),
        in_specs=[
            pl.BlockSpec((R, IN_DIM), lambda i: (i, 0)),
            pl.BlockSpec((IN_DIM, H), lambda i: (0, 0)),
            pl.BlockSpec((1, H), lambda i: (0, 0)),
        ],
        out_specs=[
            pl.BlockSpec((R, H), lambda i: (i, 0)),
            pl.BlockSpec((1, H), lambda i: (0, 0)),
        ],
        out_shape=[
            jax.ShapeDtypeStruct((N, H), _f32),
            jax.ShapeDtypeStruct((1, H), _f32),
        ],
    )(f, w, b)


def _layer_even_body(x_ref, a_ref, w1_ref, b1_ref, w2_ref, b2_ref,
                     xo_ref, p_ref):
    h = x_ref[...] + a_ref[...]
    t = jnp.maximum(_dot(h, w1_ref[...]) + b1_ref[...], 0.0)
    o = _dot(t, w2_ref[...]) + b2_ref[...]
    xo = jnp.maximum(o, 0.0)
    xo_ref[...] = xo

    @pl.when(pl.program_id(0) == 0)
    def _():
        p_ref[...] = jnp.zeros_like(p_ref)

    p_ref[...] += jnp.sum(xo, axis=0, keepdims=True)


def _layer_odd_body(x_ref, a_ref, r_ref, w1_ref, b1_ref, w2_ref, b2_ref,
                    xo_ref, ro_ref, p_ref):
    h = x_ref[...] + a_ref[...]
    t = jnp.maximum(_dot(h, w1_ref[...]) + b1_ref[...], 0.0)
    o = _dot(t, w2_ref[...]) + b2_ref[...] + r_ref[...]
    ro_ref[...] = o
    xo = jnp.maximum(o, 0.0)
    xo_ref[...] = xo

    @pl.when(pl.program_id(0) == 0)
    def _():
        p_ref[...] = jnp.zeros_like(p_ref)

    p_ref[...] += jnp.sum(xo, axis=0, keepdims=True)


_NODE_SPEC = pl.BlockSpec((R, H), lambda i: (i, 0))
_W_SPEC = pl.BlockSpec((H, H), lambda i: (0, 0))
_B_SPEC = pl.BlockSpec((1, H), lambda i: (0, 0))


def _tc_layer_even(x, agg, w1, b1, w2, b2):
    return pl.pallas_call(
        _layer_even_body,
        grid=(GRID,),
        in_specs=[_NODE_SPEC, _NODE_SPEC, _W_SPEC, _B_SPEC, _W_SPEC, _B_SPEC],
        out_specs=[_NODE_SPEC, _B_SPEC],
        out_shape=[
            jax.ShapeDtypeStruct((N, H), _f32),
            jax.ShapeDtypeStruct((1, H), _f32),
        ],
    )(x, agg, w1, b1, w2, b2)


def _tc_layer_odd(x, agg, res, w1, b1, w2, b2):
    return pl.pallas_call(
        _layer_odd_body,
        grid=(GRID,),
        in_specs=[_NODE_SPEC, _NODE_SPEC, _NODE_SPEC, _W_SPEC, _B_SPEC,
                  _W_SPEC, _B_SPEC],
        out_specs=[_NODE_SPEC, _NODE_SPEC, _B_SPEC],
        out_shape=[
            jax.ShapeDtypeStruct((N, H), _f32),
            jax.ShapeDtypeStruct((N, H), _f32),
            jax.ShapeDtypeStruct((1, H), _f32),
        ],
    )(x, agg, res, w1, b1, w2, b2)


def _head_body(p1_ref, p2_ref, wp1_ref, bp1_ref, wp2_ref, bp2_ref, hb_ref,
               s_ref, g_ref):
    def mlp(p):
        t = jnp.maximum(_dot(p, wp1_ref[...]) + bp1_ref[...], 0.0)
        return _dot(t, wp2_ref[...]) + bp2_ref[...]

    d = mlp(p1_ref[...]) - mlp(p2_ref[...])
    s = jnp.sqrt(jnp.sum(d * d))
    s_ref[0, 0] = s
    g_ref[0, 0] = s * hb_ref[0, 0]


def _tc_head(p1, p2, wp1, bp1, wp2, bp2, hb_arr):
    cat = H * (N_LAYERS + 1)
    return pl.pallas_call(
        _head_body,
        in_specs=[
            pl.BlockSpec((1, cat), lambda: (0, 0)),
            pl.BlockSpec((1, cat), lambda: (0, 0)),
            pl.BlockSpec((cat, H), lambda: (0, 0)),
            pl.BlockSpec((1, H), lambda: (0, 0)),
            pl.BlockSpec((H, H), lambda: (0, 0)),
            pl.BlockSpec((1, H), lambda: (0, 0)),
            pl.BlockSpec(memory_space=pltpu.SMEM),
        ],
        out_specs=[
            pl.BlockSpec(memory_space=pltpu.SMEM),
            pl.BlockSpec(memory_space=pltpu.SMEM),
        ],
        out_shape=[
            jax.ShapeDtypeStruct((1, 1), _f32),
            jax.ShapeDtypeStruct((1, 1), _f32),
        ],
    )(p1, p2, wp1, bp1, wp2, bp2, hb_arr)


def kernel(features_1, edge_index_1, features_2, edge_index_2, hb, W_pre,
           b_pre, conv_W1, conv_b1, conv_W2, conv_b2, Wp1, bp1, Wp2, bp2):
    b_pre2 = b_pre.reshape(1, H)
    bp1_2 = bp1.reshape(1, H)
    bp2_2 = bp2.reshape(1, H)

    prep1 = _prep_edges(edge_index_1)
    prep2 = _prep_edges(edge_index_2)

    x1, p1_0 = _tc_pre(features_1, W_pre, b_pre2)
    x2, p2_0 = _tc_pre(features_2, W_pre, b_pre2)
    res1, res2 = x1, x2
    pooled1, pooled2 = [p1_0], [p2_0]

    for i in range(N_LAYERS):
        w1 = conv_W1[i]
        b1 = conv_b1[i].reshape(1, H)
        w2 = conv_W2[i]
        b2 = conv_b2[i].reshape(1, H)
        agg1 = _sc_scatter(x1, *prep1)
        agg2 = _sc_scatter(x2, *prep2)
        if i & 1:
            x1, res1, p1 = _tc_layer_odd(x1, agg1, res1, w1, b1, w2, b2)
            x2, res2, p2 = _tc_layer_odd(x2, agg2, res2, w1, b1, w2, b2)
        else:
            x1, p1 = _tc_layer_even(x1, agg1, w1, b1, w2, b2)
            x2, p2 = _tc_layer_even(x2, agg2, w1, b1, w2, b2)
        pooled1.append(p1)
        pooled2.append(p2)

    pc1 = jnp.concatenate(pooled1, axis=1)
    pc2 = jnp.concatenate(pooled2, axis=1)
    hb_arr = jnp.asarray(hb, _f32).reshape(1, 1)
    s11, g11 = _tc_head(pc1, pc2, Wp1, bp1_2, Wp2, bp2_2, hb_arr)
    return (s11.reshape(-1), g11.reshape(-1))
